# baseline (speedup 1.0000x reference)
import functools

import jax
import jax.numpy as jnp
from jax import lax
from jax.experimental import pallas as pl
from jax.experimental.pallas import tpu as pltpu
from jax.experimental.pallas import tpu_sc as plsc

T = 2048
D = 768
E = 8
H = 1536
NE = E + 1
NA = 2 * T
TILE_M = 128
N_TILES = NA // TILE_M
MAX_STEPS = N_TILES + E
LANES = 128

NWORK = 32
CH = T // NWORK
HCH = CH // 2
NV = D // 16


def _iround(v):
    return (v + 0.5).astype(jnp.int32)


def _fround(v):
    return _iround(v).astype(jnp.float32)


def _router_body(x_ref, rw_ref, p0_ref, p1_ref, w0_ref, w1_ref, meta_ref):
    xf = x_ref[...]
    logits = lax.dot_general(
        xf, rw_ref[...], dimension_numbers=(((1,), (1,)), ((), ())),
        preferred_element_type=jnp.float32)
    lane = lax.broadcasted_iota(jnp.int32, (T, LANES), 1)
    valid = lane < E
    ml = jnp.where(valid, logits, -1e30)
    m = jnp.max(ml, axis=1, keepdims=True)
    ex = jnp.where(valid, jnp.exp(ml - m), 0.0)
    p = ex / jnp.sum(ex, axis=1, keepdims=True)
    w1 = jnp.max(p, axis=1, keepdims=True)
    i1 = jnp.min(jnp.where(p == w1, lane, LANES), axis=1, keepdims=True)
    pm = jnp.where(lane == i1, -1.0, p)
    w2 = jnp.max(pm, axis=1, keepdims=True)
    i2 = jnp.min(jnp.where(pm == w2, lane, LANES), axis=1, keepdims=True)

    oh = (lane == i1).astype(jnp.float32) + (lane == i2).astype(jnp.float32)
    stl_t = (lax.broadcasted_iota(jnp.int32, (T, T), 1)
             < lax.broadcasted_iota(jnp.int32, (T, T), 0)).astype(jnp.float32)
    cex = _fround(lax.dot_general(
        stl_t, oh, dimension_numbers=(((1,), (0,)), ((), ())),
        preferred_element_type=jnp.float32))
    cnt = jnp.sum(oh, axis=0, keepdims=True)
    lane_r = lax.broadcasted_iota(jnp.int32, (1, LANES), 1)

    ea = lax.broadcasted_iota(jnp.int32, (LANES, LANES), 0)
    eb = lax.broadcasted_iota(jnp.int32, (LANES, LANES), 1)
    m_strict = (ea < eb).astype(jnp.float32)
    m_incl = (ea <= eb).astype(jnp.float32)
    ident = (ea == eb).astype(jnp.float32)

    def mm(a, b):
        return lax.dot_general(a, b, dimension_numbers=(((1,), (0,)), ((), ())),
                               preferred_element_type=jnp.float32)

    def mmx(a, b):
        return lax.dot_general(a, b, dimension_numbers=(((1,), (0,)), ((), ())),
                               preferred_element_type=jnp.float32,
                               precision=lax.Precision.HIGHEST)

    off_row = _fround(mmx(cnt, m_strict))

    rank0 = jnp.sum(jnp.where(lane == i1, cex, 0.0), axis=1, keepdims=True)
    rank1 = jnp.sum(jnp.where(lane == i2, cex, 0.0), axis=1, keepdims=True)
    off0 = jnp.sum(jnp.where(lane == i1, off_row, 0.0), axis=1, keepdims=True)
    off1 = jnp.sum(jnp.where(lane == i2, off_row, 0.0), axis=1, keepdims=True)
    p0_ref[...] = _iround(off0 + rank0)
    p1_ref[...] = _iround(off1 + rank1)
    w0_ref[...] = jnp.broadcast_to(w1, (T, LANES))
    w1_ref[...] = jnp.broadcast_to(w2, (T, LANES))

    off_col = lax.dot_general(ident, off_row,
                              dimension_numbers=(((1,), (1,)), ((), ())),
                              preferred_element_type=jnp.float32,
                              precision=lax.Precision.HIGHEST)
    tstart = lax.broadcasted_iota(
        jnp.int32, (LANES, LANES), 1).astype(jnp.float32) * TILE_M
    firste = jnp.sum((off_col <= tstart).astype(jnp.float32),
                     axis=0, keepdims=True) - 1.0
    laste = jnp.sum((off_col <= tstart + (TILE_M - 1)).astype(jnp.float32),
                    axis=0, keepdims=True) - 1.0
    tl = lax.broadcasted_iota(jnp.int32, (1, LANES), 1)
    is_tile = tl < N_TILES
    visits = jnp.where(is_tile, laste - firste + 1.0, 0.0)
    cumv_in = _fround(mm(visits, m_incl))
    cumv_ex = _fround(mm(visits, m_strict))
    total = jnp.sum(jnp.where(tl == N_TILES - 1, cumv_in, 0.0))

    s_mat = lax.broadcasted_iota(
        jnp.int32, (LANES, LANES), 0).astype(jnp.float32)
    fin = (jnp.broadcast_to(cumv_in, (LANES, LANES)) <= s_mat) & \
          jnp.broadcast_to(is_tile, (LANES, LANES))
    tid_col = jnp.minimum(
        jnp.sum(fin.astype(jnp.float32), axis=1, keepdims=True),
        float(N_TILES - 1))
    tmatch = (lax.broadcasted_iota(
        jnp.int32, (LANES, LANES), 1).astype(jnp.float32) == tid_col)
    fsel = jnp.sum(jnp.where(tmatch, firste, 0.0), axis=1, keepdims=True)
    lsel = jnp.sum(jnp.where(tmatch, laste, 0.0), axis=1, keepdims=True)
    cxsel = jnp.sum(jnp.where(tmatch, cumv_ex, 0.0), axis=1, keepdims=True)
    s_lin = lax.broadcasted_iota(jnp.int32, (LANES, 1), 0).astype(jnp.float32)
    eid_col = jnp.clip(fsel + s_lin - cxsel, 0.0, lsel)
    valid_col = (s_lin < total).astype(jnp.float32)

    def trow(col):
        return lax.dot_general(col, ident,
                               dimension_numbers=(((0,), (0,)), ((), ())),
                               preferred_element_type=jnp.float32)

    tid_row = _fround(trow(tid_col))
    eid_row = trow(eid_col)
    valid_row = trow(valid_col)
    shift = (ea == eb - 1).astype(jnp.float32)
    prev_row = mm(tid_row, shift)
    first_row = valid_row * jnp.where(
        (lane_r == 0) | (tid_row != prev_row), 1.0, 0.0)

    meta_ref[0:1, :] = _iround(tid_row)
    meta_ref[1:2, :] = _iround(eid_row)
    meta_ref[2:3, :] = _iround(valid_row)
    meta_ref[3:4, :] = _iround(off_row)
    meta_ref[4:5, :] = _iround(first_row)
    meta_ref[5:8, :] = jnp.zeros((3, LANES), jnp.int32)


def _gmm_body(offm_ref, tid_ref, eid_ref, vld_ref, fst_ref,
              xs_ref, wfc_ref, wg_ref, wp_ref, ws_ref, o_ref):
    s = pl.program_id(0)

    @pl.when(vld_ref[s] == 1)
    def _():
        e = eid_ref[s]
        xb = xs_ref[...]
        h = lax.dot_general(xb, wfc_ref[0],
                            dimension_numbers=(((1,), (1,)), ((), ())),
                            preferred_element_type=jnp.float32)
        g = lax.dot_general(xb, wg_ref[0],
                            dimension_numbers=(((1,), (1,)), ((), ())),
                            preferred_element_type=jnp.float32)
        a = h * (g * jax.nn.sigmoid(g))
        o = lax.dot_general(a, wp_ref[0],
                            dimension_numbers=(((1,), (1,)), ((), ())),
                            preferred_element_type=jnp.float32)
        rows = (tid_ref[s] * TILE_M
                + lax.broadcasted_iota(jnp.int32, (TILE_M, 1), 0))
        lo = offm_ref[e]
        hi = offm_ref[e + 1]
        weff = jnp.where((rows >= lo) & (rows < hi), ws_ref[...][:, 0:1], 0.0)
        ow = o * weff

        @pl.when(fst_ref[s] == 1)
        def _():
            o_ref[...] = ow

        @pl.when(fst_ref[s] == 0)
        def _():
            o_ref[...] += ow


def _dispatch_body(x_hbm, p0_hbm, p1_hbm, w0_hbm, w1_hbm,
                   xs_hbm, ws_hbm, rows_v, i0_v, i1_v, w0_v, w1_v, sem):
    wid = lax.axis_index("s") * 2 + lax.axis_index("c")
    base = wid * CH
    pltpu.sync_copy(x_hbm.at[pl.ds(base, CH)], rows_v)
    pltpu.sync_copy(p0_hbm.at[pl.ds(base, CH)], i0_v)
    c0 = pltpu.async_copy(rows_v, xs_hbm.at[i0_v], sem)
    pltpu.sync_copy(p1_hbm.at[pl.ds(base, CH)], i1_v)
    c1 = pltpu.async_copy(rows_v, xs_hbm.at[i1_v], sem)
    pltpu.sync_copy(w0_hbm.at[pl.ds(base, CH)], w0_v)
    c2 = pltpu.async_copy(w0_v, ws_hbm.at[i0_v], sem)
    pltpu.sync_copy(w1_hbm.at[pl.ds(base, CH)], w1_v)
    c3 = pltpu.async_copy(w1_v, ws_hbm.at[i1_v], sem)
    c0.wait()
    c1.wait()
    c2.wait()
    c3.wait()


QCH = 16
NQ = CH // QCH


def _combine_body(o_hbm, ys_hbm, p0_hbm, p1_hbm, y_hbm, g0_v, g1_v, g2_v,
                  i0_v, i1_v, sem0, sem1):
    wid = lax.axis_index("s") * 2 + lax.axis_index("c")
    base = wid * CH
    pltpu.sync_copy(p0_hbm.at[pl.ds(base, CH)], i0_v)
    pltpu.sync_copy(p1_hbm.at[pl.ds(base, CH)], i1_v)
    sems = (sem0, sem1)

    def fire(q, b):
        sl = pl.ds(q * QCH, QCH)
        lin = pl.ds(base + q * QCH, QCH)
        return (pltpu.async_copy(o_hbm.at[i0_v.at[sl]], g0_v.at[b], sems[b]),
                pltpu.async_copy(o_hbm.at[i1_v.at[sl]], g1_v.at[b], sems[b]),
                pltpu.async_copy(ys_hbm.at[lin], g2_v.at[b], sems[b]))

    inflight = fire(0, 0)
    for q in range(NQ):
        b = q % 2
        for c in inflight:
            c.wait()
        if q + 1 < NQ:
            inflight = fire(q + 1, (q + 1) % 2)

        def rbody(r, carry):
            for c in range(NV):
                sl = pl.ds(c * 16, 16)
                g0_v[b, r, sl] = g0_v[b, r, sl] + g1_v[b, r, sl] + g2_v[b, r, sl]
            return carry

        lax.fori_loop(0, QCH, rbody, 0)
        pltpu.sync_copy(g0_v.at[b], y_hbm.at[pl.ds(base + q * QCH, QCH)])


def _dispatch(x_flat, p0, p1, w0x, w1x):
    mesh = plsc.VectorSubcoreMesh(core_axis_name="c", subcore_axis_name="s")
    run = functools.partial(
        pl.kernel, mesh=mesh,
        out_type=[jax.ShapeDtypeStruct((NA, D), jnp.float32),
                  jax.ShapeDtypeStruct((NA, LANES), jnp.float32)],
        scratch_types=[pltpu.VMEM((CH, D), jnp.float32),
                       pltpu.VMEM((CH,), jnp.int32),
                       pltpu.VMEM((CH,), jnp.int32),
                       pltpu.VMEM((CH, LANES), jnp.float32),
                       pltpu.VMEM((CH, LANES), jnp.float32),
                       pltpu.SemaphoreType.DMA])(_dispatch_body)
    return run(x_flat, p0, p1, w0x, w1x)


def _combine(o_sorted, ys, p0, p1):
    mesh = plsc.VectorSubcoreMesh(core_axis_name="c", subcore_axis_name="s")
    run = functools.partial(
        pl.kernel, mesh=mesh,
        out_type=jax.ShapeDtypeStruct((T, D), jnp.float32),
        scratch_types=[pltpu.VMEM((2, QCH, D), jnp.float32),
                       pltpu.VMEM((2, QCH, D), jnp.float32),
                       pltpu.VMEM((2, QCH, D), jnp.float32),
                       pltpu.VMEM((CH,), jnp.int32),
                       pltpu.VMEM((CH,), jnp.int32),
                       pltpu.SemaphoreType.DMA,
                       pltpu.SemaphoreType.DMA])(_combine_body)
    return run(o_sorted, ys, p0, p1)


def _shared_body(x_ref, wfc_ref, wg_ref, wp_ref, y_ref):
    xb = x_ref[...]
    h = lax.dot_general(xb, wfc_ref[...],
                        dimension_numbers=(((1,), (1,)), ((), ())),
                        preferred_element_type=jnp.float32)
    g = lax.dot_general(xb, wg_ref[...],
                        dimension_numbers=(((1,), (1,)), ((), ())),
                        preferred_element_type=jnp.float32)
    a = h * (g * jax.nn.sigmoid(g))
    y_ref[...] = lax.dot_general(a, wp_ref[...],
                                 dimension_numbers=(((1,), (1,)), ((), ())),
                                 preferred_element_type=jnp.float32)


def _shared_call(x_flat, wfc_s, wg_s, wp_s):
    return pl.pallas_call(
        _shared_body,
        grid=(T // TILE_M,),
        in_specs=[
            pl.BlockSpec((TILE_M, D), lambda t: (t, 0)),
            pl.BlockSpec((H, D), lambda t: (0, 0)),
            pl.BlockSpec((H, D), lambda t: (0, 0)),
            pl.BlockSpec((D, H), lambda t: (0, 0)),
        ],
        out_specs=pl.BlockSpec((TILE_M, D), lambda t: (t, 0)),
        out_shape=jax.ShapeDtypeStruct((T, D), jnp.float32),
    )(x_flat, wfc_s, wg_s, wp_s)


def _gmm_call(offm, tid, eid, vld, fst, xs, wfc_all, wg_all, wp_all, ws):
    grid_spec = pltpu.PrefetchScalarGridSpec(
        num_scalar_prefetch=5,
        grid=(MAX_STEPS,),
        in_specs=[
            pl.BlockSpec((TILE_M, D), lambda s, om, ti, ei, vl, fs: (ti[s], 0)),
            pl.BlockSpec((1, H, D), lambda s, om, ti, ei, vl, fs: (ei[s], 0, 0)),
            pl.BlockSpec((1, H, D), lambda s, om, ti, ei, vl, fs: (ei[s], 0, 0)),
            pl.BlockSpec((1, D, H), lambda s, om, ti, ei, vl, fs: (ei[s], 0, 0)),
            pl.BlockSpec((TILE_M, LANES),
                         lambda s, om, ti, ei, vl, fs: (ti[s], 0)),
        ],
        out_specs=pl.BlockSpec((TILE_M, D),
                               lambda s, om, ti, ei, vl, fs: (ti[s], 0)),
    )
    return pl.pallas_call(
        _gmm_body,
        grid_spec=grid_spec,
        out_shape=jax.ShapeDtypeStruct((NA, D), jnp.float32),
    )(offm, tid, eid, vld, fst, xs, wfc_all, wg_all, wp_all, ws)


def _router_call(x_flat, rw_pad):
    return pl.pallas_call(
        _router_body,
        out_shape=[
            jax.ShapeDtypeStruct((T, 1), jnp.int32),
            jax.ShapeDtypeStruct((T, 1), jnp.int32),
            jax.ShapeDtypeStruct((T, LANES), jnp.float32),
            jax.ShapeDtypeStruct((T, LANES), jnp.float32),
            jax.ShapeDtypeStruct((8, LANES), jnp.int32),
        ],
    )(x_flat, rw_pad)


def kernel(x, router_W, Wfc, Wgate, Wproj, Wfc_s, Wgate_s, Wproj_s):
    x_flat = x.reshape(T, D)
    rw_pad = jnp.zeros((LANES, D), jnp.float32).at[:E].set(router_W)

    p0c, p1c, w0x, w1x, meta = _router_call(x_flat, rw_pad)
    p0 = p0c.reshape(T)
    p1 = p1c.reshape(T)
    tid = meta[0, :MAX_STEPS]
    eid = meta[1, :MAX_STEPS]
    vld = meta[2, :MAX_STEPS]
    offm = meta[3, :16]
    fst = meta[4, :MAX_STEPS]

    xs, ws = _dispatch(x_flat, p0, p1, w0x, w1x)
    ys = _shared_call(x_flat, Wfc_s, Wgate_s, Wproj_s)
    o_sorted = _gmm_call(offm, tid, eid, vld, fst,
                         xs, Wfc, Wgate, Wproj, ws)
    y = _combine(o_sorted, ys, p0, p1)
    return y.reshape(1, T, D)


# speedup vs baseline: 1.5090x; 1.5090x over previous
import functools

import jax
import jax.numpy as jnp
from jax import lax
from jax.experimental import pallas as pl
from jax.experimental.pallas import tpu as pltpu
from jax.experimental.pallas import tpu_sc as plsc

T = 2048
D = 768
E = 8
H = 1536
NE = E + 1
NA = 2 * T
TILE_M = 512
N_TILES = NA // TILE_M
MAX_STEPS = N_TILES + E
LANES = 128

NWORK = 32
CH = T // NWORK
HCH = CH // 2
NV = D // 16


def _iround(v):
    return (v + 0.5).astype(jnp.int32)


def _fround(v):
    return _iround(v).astype(jnp.float32)


def _router_body(x_ref, rw_ref, p0_ref, p1_ref, w0_ref, w1_ref, meta_ref):
    xf = x_ref[...]
    logits = lax.dot_general(
        xf, rw_ref[...], dimension_numbers=(((1,), (1,)), ((), ())),
        preferred_element_type=jnp.float32)
    lane = lax.broadcasted_iota(jnp.int32, (T, LANES), 1)
    valid = lane < E
    ml = jnp.where(valid, logits, -1e30)
    m = jnp.max(ml, axis=1, keepdims=True)
    ex = jnp.where(valid, jnp.exp(ml - m), 0.0)
    p = ex / jnp.sum(ex, axis=1, keepdims=True)
    w1 = jnp.max(p, axis=1, keepdims=True)
    i1 = jnp.min(jnp.where(p == w1, lane, LANES), axis=1, keepdims=True)
    pm = jnp.where(lane == i1, -1.0, p)
    w2 = jnp.max(pm, axis=1, keepdims=True)
    i2 = jnp.min(jnp.where(pm == w2, lane, LANES), axis=1, keepdims=True)

    oh = (lane == i1).astype(jnp.float32) + (lane == i2).astype(jnp.float32)
    stl_t = (lax.broadcasted_iota(jnp.int32, (T, T), 1)
             < lax.broadcasted_iota(jnp.int32, (T, T), 0)).astype(jnp.float32)
    cex = _fround(lax.dot_general(
        stl_t, oh, dimension_numbers=(((1,), (0,)), ((), ())),
        preferred_element_type=jnp.float32))
    cnt = jnp.sum(oh, axis=0, keepdims=True)
    lane_r = lax.broadcasted_iota(jnp.int32, (1, LANES), 1)

    ea = lax.broadcasted_iota(jnp.int32, (LANES, LANES), 0)
    eb = lax.broadcasted_iota(jnp.int32, (LANES, LANES), 1)
    m_strict = (ea < eb).astype(jnp.float32)
    m_incl = (ea <= eb).astype(jnp.float32)
    ident = (ea == eb).astype(jnp.float32)

    def mm(a, b):
        return lax.dot_general(a, b, dimension_numbers=(((1,), (0,)), ((), ())),
                               preferred_element_type=jnp.float32)

    def mmx(a, b):
        return lax.dot_general(a, b, dimension_numbers=(((1,), (0,)), ((), ())),
                               preferred_element_type=jnp.float32,
                               precision=lax.Precision.HIGHEST)

    off_row = _fround(mmx(cnt, m_strict))

    rank0 = jnp.sum(jnp.where(lane == i1, cex, 0.0), axis=1, keepdims=True)
    rank1 = jnp.sum(jnp.where(lane == i2, cex, 0.0), axis=1, keepdims=True)
    off0 = jnp.sum(jnp.where(lane == i1, off_row, 0.0), axis=1, keepdims=True)
    off1 = jnp.sum(jnp.where(lane == i2, off_row, 0.0), axis=1, keepdims=True)
    p0_ref[...] = _iround(off0 + rank0)
    p1_ref[...] = _iround(off1 + rank1)
    w0_ref[...] = jnp.broadcast_to(w1, (T, LANES))
    w1_ref[...] = jnp.broadcast_to(w2, (T, LANES))

    off_col = lax.dot_general(ident, off_row,
                              dimension_numbers=(((1,), (1,)), ((), ())),
                              preferred_element_type=jnp.float32,
                              precision=lax.Precision.HIGHEST)
    tstart = lax.broadcasted_iota(
        jnp.int32, (LANES, LANES), 1).astype(jnp.float32) * TILE_M
    firste = jnp.sum((off_col <= tstart).astype(jnp.float32),
                     axis=0, keepdims=True) - 1.0
    laste = jnp.sum((off_col <= tstart + (TILE_M - 1)).astype(jnp.float32),
                    axis=0, keepdims=True) - 1.0
    tl = lax.broadcasted_iota(jnp.int32, (1, LANES), 1)
    is_tile = tl < N_TILES
    visits = jnp.where(is_tile, laste - firste + 1.0, 0.0)
    cumv_in = _fround(mm(visits, m_incl))
    cumv_ex = _fround(mm(visits, m_strict))
    total = jnp.sum(jnp.where(tl == N_TILES - 1, cumv_in, 0.0))

    s_mat = lax.broadcasted_iota(
        jnp.int32, (LANES, LANES), 0).astype(jnp.float32)
    fin = (jnp.broadcast_to(cumv_in, (LANES, LANES)) <= s_mat) & \
          jnp.broadcast_to(is_tile, (LANES, LANES))
    tid_col = jnp.minimum(
        jnp.sum(fin.astype(jnp.float32), axis=1, keepdims=True),
        float(N_TILES - 1))
    tmatch = (lax.broadcasted_iota(
        jnp.int32, (LANES, LANES), 1).astype(jnp.float32) == tid_col)
    fsel = jnp.sum(jnp.where(tmatch, firste, 0.0), axis=1, keepdims=True)
    lsel = jnp.sum(jnp.where(tmatch, laste, 0.0), axis=1, keepdims=True)
    cxsel = jnp.sum(jnp.where(tmatch, cumv_ex, 0.0), axis=1, keepdims=True)
    s_lin = lax.broadcasted_iota(jnp.int32, (LANES, 1), 0).astype(jnp.float32)
    eid_col = jnp.clip(fsel + s_lin - cxsel, 0.0, lsel)
    valid_col = (s_lin < total).astype(jnp.float32)

    def trow(col):
        return lax.dot_general(col, ident,
                               dimension_numbers=(((0,), (0,)), ((), ())),
                               preferred_element_type=jnp.float32)

    tid_row = _fround(trow(tid_col))
    eid_row = trow(eid_col)
    valid_row = trow(valid_col)
    shift = (ea == eb - 1).astype(jnp.float32)
    prev_row = mm(tid_row, shift)
    first_row = valid_row * jnp.where(
        (lane_r == 0) | (tid_row != prev_row), 1.0, 0.0)

    meta_ref[0:1, :] = _iround(tid_row)
    meta_ref[1:2, :] = _iround(eid_row)
    meta_ref[2:3, :] = _iround(valid_row)
    meta_ref[3:4, :] = _iround(off_row)
    meta_ref[4:5, :] = _iround(first_row)
    meta_ref[5:8, :] = jnp.zeros((3, LANES), jnp.int32)


def _gmm_body(offm_ref, tid_ref, eid_ref, vld_ref, fst_ref,
              xs_ref, wfc_ref, wg_ref, wp_ref, ws_ref, o_ref):
    s = pl.program_id(0)

    @pl.when(vld_ref[s] == 1)
    def _():
        e = eid_ref[s]
        xb = xs_ref[...]
        h = lax.dot_general(xb, wfc_ref[0],
                            dimension_numbers=(((1,), (1,)), ((), ())),
                            preferred_element_type=jnp.float32)
        g = lax.dot_general(xb, wg_ref[0],
                            dimension_numbers=(((1,), (1,)), ((), ())),
                            preferred_element_type=jnp.float32)
        a = h * (g * jax.nn.sigmoid(g))
        o = lax.dot_general(a, wp_ref[0],
                            dimension_numbers=(((1,), (1,)), ((), ())),
                            preferred_element_type=jnp.float32)
        rows = (tid_ref[s] * TILE_M
                + lax.broadcasted_iota(jnp.int32, (TILE_M, 1), 0))
        lo = offm_ref[e]
        hi = offm_ref[e + 1]
        weff = jnp.where((rows >= lo) & (rows < hi), ws_ref[...][:, 0:1], 0.0)
        ow = o * weff

        @pl.when(fst_ref[s] == 1)
        def _():
            o_ref[...] = ow

        @pl.when(fst_ref[s] == 0)
        def _():
            o_ref[...] += ow


def _dispatch_body(x_hbm, p0_hbm, p1_hbm, w0_hbm, w1_hbm,
                   xs_hbm, ws_hbm, rows_v, i0_v, i1_v, w0_v, w1_v, sem):
    wid = lax.axis_index("s") * 2 + lax.axis_index("c")
    base = wid * CH
    pltpu.sync_copy(x_hbm.at[pl.ds(base, CH)], rows_v)
    pltpu.sync_copy(p0_hbm.at[pl.ds(base, CH)], i0_v)
    c0 = pltpu.async_copy(rows_v, xs_hbm.at[i0_v], sem)
    pltpu.sync_copy(p1_hbm.at[pl.ds(base, CH)], i1_v)
    c1 = pltpu.async_copy(rows_v, xs_hbm.at[i1_v], sem)
    pltpu.sync_copy(w0_hbm.at[pl.ds(base, CH)], w0_v)
    c2 = pltpu.async_copy(w0_v, ws_hbm.at[i0_v], sem)
    pltpu.sync_copy(w1_hbm.at[pl.ds(base, CH)], w1_v)
    c3 = pltpu.async_copy(w1_v, ws_hbm.at[i1_v], sem)
    c0.wait()
    c1.wait()
    c2.wait()
    c3.wait()


QCH = 16
NQ = CH // QCH


def _combine_body(o_hbm, ys_hbm, p0_hbm, p1_hbm, y_hbm, g0_v, g1_v, g2_v,
                  i0_v, i1_v, sem0, sem1):
    wid = lax.axis_index("s") * 2 + lax.axis_index("c")
    base = wid * CH
    pltpu.sync_copy(p0_hbm.at[pl.ds(base, CH)], i0_v)
    pltpu.sync_copy(p1_hbm.at[pl.ds(base, CH)], i1_v)
    sems = (sem0, sem1)

    def fire(q, b):
        sl = pl.ds(q * QCH, QCH)
        lin = pl.ds(base + q * QCH, QCH)
        return (pltpu.async_copy(o_hbm.at[i0_v.at[sl]], g0_v.at[b], sems[b]),
                pltpu.async_copy(o_hbm.at[i1_v.at[sl]], g1_v.at[b], sems[b]),
                pltpu.async_copy(ys_hbm.at[lin], g2_v.at[b], sems[b]))

    inflight = fire(0, 0)
    for q in range(NQ):
        b = q % 2
        for c in inflight:
            c.wait()
        if q + 1 < NQ:
            inflight = fire(q + 1, (q + 1) % 2)

        def rbody(r, carry):
            for c in range(NV):
                sl = pl.ds(c * 16, 16)
                g0_v[b, r, sl] = g0_v[b, r, sl] + g1_v[b, r, sl] + g2_v[b, r, sl]
            return carry

        lax.fori_loop(0, QCH, rbody, 0)
        pltpu.sync_copy(g0_v.at[b], y_hbm.at[pl.ds(base + q * QCH, QCH)])


def _dispatch(x_flat, p0, p1, w0x, w1x):
    mesh = plsc.VectorSubcoreMesh(core_axis_name="c", subcore_axis_name="s")
    run = functools.partial(
        pl.kernel, mesh=mesh,
        out_type=[jax.ShapeDtypeStruct((NA, D), jnp.float32),
                  jax.ShapeDtypeStruct((NA, LANES), jnp.float32)],
        scratch_types=[pltpu.VMEM((CH, D), jnp.float32),
                       pltpu.VMEM((CH,), jnp.int32),
                       pltpu.VMEM((CH,), jnp.int32),
                       pltpu.VMEM((CH, LANES), jnp.float32),
                       pltpu.VMEM((CH, LANES), jnp.float32),
                       pltpu.SemaphoreType.DMA])(_dispatch_body)
    return run(x_flat, p0, p1, w0x, w1x)


def _combine(o_sorted, ys, p0, p1):
    mesh = plsc.VectorSubcoreMesh(core_axis_name="c", subcore_axis_name="s")
    run = functools.partial(
        pl.kernel, mesh=mesh,
        out_type=jax.ShapeDtypeStruct((T, D), jnp.float32),
        scratch_types=[pltpu.VMEM((2, QCH, D), jnp.float32),
                       pltpu.VMEM((2, QCH, D), jnp.float32),
                       pltpu.VMEM((2, QCH, D), jnp.float32),
                       pltpu.VMEM((CH,), jnp.int32),
                       pltpu.VMEM((CH,), jnp.int32),
                       pltpu.SemaphoreType.DMA,
                       pltpu.SemaphoreType.DMA])(_combine_body)
    return run(o_sorted, ys, p0, p1)


def _shared_body(x_ref, wfc_ref, wg_ref, wp_ref, y_ref):
    xb = x_ref[...]
    h = lax.dot_general(xb, wfc_ref[...],
                        dimension_numbers=(((1,), (1,)), ((), ())),
                        preferred_element_type=jnp.float32)
    g = lax.dot_general(xb, wg_ref[...],
                        dimension_numbers=(((1,), (1,)), ((), ())),
                        preferred_element_type=jnp.float32)
    a = h * (g * jax.nn.sigmoid(g))
    y_ref[...] = lax.dot_general(a, wp_ref[...],
                                 dimension_numbers=(((1,), (1,)), ((), ())),
                                 preferred_element_type=jnp.float32)


def _shared_call(x_flat, wfc_s, wg_s, wp_s):
    return pl.pallas_call(
        _shared_body,
        grid=(T // TILE_M,),
        in_specs=[
            pl.BlockSpec((TILE_M, D), lambda t: (t, 0)),
            pl.BlockSpec((H, D), lambda t: (0, 0)),
            pl.BlockSpec((H, D), lambda t: (0, 0)),
            pl.BlockSpec((D, H), lambda t: (0, 0)),
        ],
        out_specs=pl.BlockSpec((TILE_M, D), lambda t: (t, 0)),
        out_shape=jax.ShapeDtypeStruct((T, D), jnp.float32),
    )(x_flat, wfc_s, wg_s, wp_s)


def _gmm_call(offm, tid, eid, vld, fst, xs, wfc_all, wg_all, wp_all, ws):
    grid_spec = pltpu.PrefetchScalarGridSpec(
        num_scalar_prefetch=5,
        grid=(MAX_STEPS,),
        in_specs=[
            pl.BlockSpec((TILE_M, D), lambda s, om, ti, ei, vl, fs: (ti[s], 0)),
            pl.BlockSpec((1, H, D), lambda s, om, ti, ei, vl, fs: (ei[s], 0, 0)),
            pl.BlockSpec((1, H, D), lambda s, om, ti, ei, vl, fs: (ei[s], 0, 0)),
            pl.BlockSpec((1, D, H), lambda s, om, ti, ei, vl, fs: (ei[s], 0, 0)),
            pl.BlockSpec((TILE_M, LANES),
                         lambda s, om, ti, ei, vl, fs: (ti[s], 0)),
        ],
        out_specs=pl.BlockSpec((TILE_M, D),
                               lambda s, om, ti, ei, vl, fs: (ti[s], 0)),
    )
    return pl.pallas_call(
        _gmm_body,
        grid_spec=grid_spec,
        out_shape=jax.ShapeDtypeStruct((NA, D), jnp.float32),
    )(offm, tid, eid, vld, fst, xs, wfc_all, wg_all, wp_all, ws)


def _router_call(x_flat, rw_pad):
    return pl.pallas_call(
        _router_body,
        out_shape=[
            jax.ShapeDtypeStruct((T, 1), jnp.int32),
            jax.ShapeDtypeStruct((T, 1), jnp.int32),
            jax.ShapeDtypeStruct((T, LANES), jnp.float32),
            jax.ShapeDtypeStruct((T, LANES), jnp.float32),
            jax.ShapeDtypeStruct((8, LANES), jnp.int32),
        ],
    )(x_flat, rw_pad)


def kernel(x, router_W, Wfc, Wgate, Wproj, Wfc_s, Wgate_s, Wproj_s):
    x_flat = x.reshape(T, D)
    rw_pad = jnp.zeros((LANES, D), jnp.float32).at[:E].set(router_W)

    p0c, p1c, w0x, w1x, meta = _router_call(x_flat, rw_pad)
    p0 = p0c.reshape(T)
    p1 = p1c.reshape(T)
    tid = meta[0, :MAX_STEPS]
    eid = meta[1, :MAX_STEPS]
    vld = meta[2, :MAX_STEPS]
    offm = meta[3, :16]
    fst = meta[4, :MAX_STEPS]

    xs, ws = _dispatch(x_flat, p0, p1, w0x, w1x)
    ys = _shared_call(x_flat, Wfc_s, Wgate_s, Wproj_s)
    o_sorted = _gmm_call(offm, tid, eid, vld, fst,
                         xs, Wfc, Wgate, Wproj, ws)
    y = _combine(o_sorted, ys, p0, p1)
    return y.reshape(1, T, D)
